# baseline (device time: 13674 ns/iter reference)
import jax
import jax.numpy as jnp
from jax import lax
from jax.experimental import pallas as pl
from jax.experimental.pallas import tpu as pltpu

N_DEV = 32
Z_SIZE = 4
Y_SIZE = 4

ROW_PARTIAL, ROW_XRECV, ROW_XSUM, ROW_ZSRC = 0, 1, 2, 6


def kernel(x):
    m_per, n = x.shape
    total_rows = N_DEV * m_per

    def body(x_ref, out_ref, acc_ref, send_sems, recv_sems, cred_sems):
        me = lax.axis_index("i")
        z = me // 8
        p = lax.rem(me, 8)
        y = p // 2
        c = lax.rem(p, 2)

        partner_x = z * 8 + y * 2 + (1 - c)

        def partner_y(k):
            y2 = lax.rem(y + k, Y_SIZE)
            c2 = (1 - c) if (k & 1) else c
            return z * 8 + y2 * 2 + c2

        def partner_z(k):
            return lax.rem(me + 8 * k, N_DEV)

        barrier = pltpu.get_barrier_semaphore()
        pl.semaphore_signal(
            barrier, inc=1, device_id=(partner_x,),
            device_id_type=pl.DeviceIdType.MESH,
        )
        for k in range(1, Y_SIZE):
            pl.semaphore_signal(
                cred_sems.at[0], inc=1, device_id=(partner_y(k),),
                device_id_type=pl.DeviceIdType.MESH,
            )
        for k in range(1, Z_SIZE):
            pl.semaphore_signal(
                cred_sems.at[1], inc=1, device_id=(partner_z(k),),
                device_id_type=pl.DeviceIdType.MESH,
            )

        acc_ref[pl.ds(ROW_PARTIAL, 1), :] = jnp.sum(
            x_ref[...].astype(jnp.float32), axis=0, keepdims=True
        )

        sends = []

        pl.semaphore_wait(barrier, 1)
        rdma = pltpu.make_async_remote_copy(
            src_ref=acc_ref.at[pl.ds(ROW_PARTIAL, 1)],
            dst_ref=acc_ref.at[pl.ds(ROW_XRECV, 1)],
            send_sem=send_sems.at[0],
            recv_sem=recv_sems.at[0],
            device_id=(partner_x,),
            device_id_type=pl.DeviceIdType.MESH,
        )
        rdma.start()
        sends.append(rdma)
        rdma.wait_recv()
        acc_ref[pl.ds(ROW_XSUM, 1), :] = (
            acc_ref[pl.ds(ROW_PARTIAL, 1), :] + acc_ref[pl.ds(ROW_XRECV, 1), :]
        )

        pl.semaphore_wait(cred_sems.at[0], Y_SIZE - 1)
        for k in range(1, Y_SIZE):
            rdma = pltpu.make_async_remote_copy(
                src_ref=acc_ref.at[pl.ds(ROW_XSUM, 1)],
                dst_ref=acc_ref.at[pl.ds(ROW_XSUM + (Y_SIZE - k), 1)],
                send_sem=send_sems.at[k],
                recv_sem=recv_sems.at[Y_SIZE - k],
                device_id=(partner_y(k),),
                device_id_type=pl.DeviceIdType.MESH,
            )
            rdma.start()
            sends.append(rdma)
        for j in range(1, Y_SIZE):
            recv = pltpu.make_async_remote_copy(
                src_ref=acc_ref.at[pl.ds(ROW_XSUM, 1)],
                dst_ref=acc_ref.at[pl.ds(ROW_XSUM + j, 1)],
                send_sem=send_sems.at[0],
                recv_sem=recv_sems.at[j],
                device_id=(partner_x,),
                device_id_type=pl.DeviceIdType.MESH,
            )
            recv.wait_recv()
        acc_ref[pl.ds(ROW_ZSRC, 1), :] = (
            acc_ref[pl.ds(ROW_XSUM, 1), :]
            + acc_ref[pl.ds(ROW_XSUM + 1, 1), :]
            + acc_ref[pl.ds(ROW_XSUM + 2, 1), :]
            + acc_ref[pl.ds(ROW_XSUM + 3, 1), :]
        )

        pl.semaphore_wait(cred_sems.at[1], Z_SIZE - 1)
        for k in range(1, Z_SIZE):
            rdma = pltpu.make_async_remote_copy(
                src_ref=acc_ref.at[pl.ds(ROW_ZSRC, 1)],
                dst_ref=acc_ref.at[pl.ds(ROW_ZSRC + (Z_SIZE - k), 1)],
                send_sem=send_sems.at[Y_SIZE - 1 + k],
                recv_sem=recv_sems.at[Y_SIZE - 1 + (Z_SIZE - k)],
                device_id=(partner_z(k),),
                device_id_type=pl.DeviceIdType.MESH,
            )
            rdma.start()
            sends.append(rdma)
        for j in range(1, Z_SIZE):
            recv = pltpu.make_async_remote_copy(
                src_ref=acc_ref.at[pl.ds(ROW_ZSRC, 1)],
                dst_ref=acc_ref.at[pl.ds(ROW_ZSRC + j, 1)],
                send_sem=send_sems.at[0],
                recv_sem=recv_sems.at[Y_SIZE - 1 + j],
                device_id=(partner_x,),
                device_id_type=pl.DeviceIdType.MESH,
            )
            recv.wait_recv()

        out_ref[...] = (
            (
                acc_ref[pl.ds(ROW_ZSRC, 1), :]
                + acc_ref[pl.ds(ROW_ZSRC + 1, 1), :]
                + acc_ref[pl.ds(ROW_ZSRC + 2, 1), :]
                + acc_ref[pl.ds(ROW_ZSRC + 3, 1), :]
            )
            * (1.0 / total_rows)
        ).astype(jnp.float32)

        for rdma in sends:
            rdma.wait_send()

    return pl.pallas_call(
        body,
        out_shape=jax.ShapeDtypeStruct((1, n), jnp.float32),
        in_specs=[pl.BlockSpec(memory_space=pltpu.VMEM)],
        out_specs=pl.BlockSpec(memory_space=pltpu.VMEM),
        scratch_shapes=[
            pltpu.VMEM((10, n), jnp.float32),
            pltpu.SemaphoreType.DMA((7,)),
            pltpu.SemaphoreType.DMA((7,)),
            pltpu.SemaphoreType.REGULAR((2,)),
        ],
        compiler_params=pltpu.CompilerParams(collective_id=0),
    )(x)


# device time: 12422 ns/iter; 1.1008x vs baseline; 1.1008x over previous
import jax
import jax.numpy as jnp
from jax import lax
from jax.experimental import pallas as pl
from jax.experimental.pallas import tpu as pltpu

N_DEV = 32


def kernel(x):
    m_per, n = x.shape
    total_rows = N_DEV * m_per

    def body(x_ref, out_ref, acc_ref, send_sems, recv_sems):
        my_i = lax.axis_index("i")

        barrier = pltpu.get_barrier_semaphore()
        for k in range(1, N_DEV):
            pl.semaphore_signal(
                barrier,
                inc=1,
                device_id=((my_i + k) % N_DEV,),
                device_id_type=pl.DeviceIdType.MESH,
            )

        acc_ref[pl.ds(0, 1), :] = jnp.sum(
            x_ref[...].astype(jnp.float32), axis=0, keepdims=True
        )

        pl.semaphore_wait(barrier, N_DEV - 1)

        sends = []
        for k in range(1, N_DEV):
            rdma = pltpu.make_async_remote_copy(
                src_ref=acc_ref.at[pl.ds(0, 1)],
                dst_ref=acc_ref.at[pl.ds(N_DEV - k, 1)],
                send_sem=send_sems.at[k],
                recv_sem=recv_sems.at[N_DEV - k],
                device_id=((my_i + k) % N_DEV,),
                device_id_type=pl.DeviceIdType.MESH,
            )
            rdma.start()
            sends.append(rdma)

        for j in range(1, N_DEV):
            recv = pltpu.make_async_remote_copy(
                src_ref=acc_ref.at[pl.ds(0, 1)],
                dst_ref=acc_ref.at[pl.ds(j, 1)],
                send_sem=send_sems.at[0],
                recv_sem=recv_sems.at[j],
                device_id=((my_i + 1) % N_DEV,),
                device_id_type=pl.DeviceIdType.MESH,
            )
            recv.wait_recv()

        out_ref[...] = (
            jnp.sum(acc_ref[...], axis=0, keepdims=True) * (1.0 / total_rows)
        ).astype(jnp.float32)

        for rdma in sends:
            rdma.wait_send()

    return pl.pallas_call(
        body,
        out_shape=jax.ShapeDtypeStruct((1, n), jnp.float32),
        in_specs=[pl.BlockSpec(memory_space=pltpu.VMEM)],
        out_specs=pl.BlockSpec(memory_space=pltpu.VMEM),
        scratch_shapes=[
            pltpu.VMEM((N_DEV, n), jnp.float32),
            pltpu.SemaphoreType.DMA((N_DEV,)),
            pltpu.SemaphoreType.DMA((N_DEV,)),
        ],
        compiler_params=pltpu.CompilerParams(collective_id=0),
    )(x)
